# baseline (speedup 1.0000x reference)
import functools
import math

import jax
import jax.numpy as jnp
from jax import lax
from jax.experimental import pallas as pl
from jax.experimental.pallas import tpu as pltpu
from jax.experimental.pallas import tpu_sc as plsc

NATOM = 128
N_RADIAL = 20
CUTOFF = 5.0

N_NODES = 10000
N_EDGES = 320000

NC = 2
NS = 16
L = 16

D = 3 * NATOM
FD = 512
XMD = 2 * D

NW = NC * NS
NPASS = 3
TR = 112
NPAD = NPASS * NW * TR
ACC_ROWS = TR + L
CHUNK = 2000
NVEC = CHUNK // L
NCHUNK = N_EDGES // CHUNK
CAP = 2048
CMASK = CAP - 1
SCU = 5


def _mlp_body(q_ref, W1_ref, b1_ref, W2_ref, b2_ref, x_ref):
    h = jnp.dot(q_ref[...], W1_ref[...], preferred_element_type=jnp.float32)
    h = h + b1_ref[...]
    h = h * jax.nn.sigmoid(h)
    x_ref[...] = (
        jnp.dot(h, W2_ref[...], preferred_element_type=jnp.float32) + b2_ref[...]
    )


def _filter_body(ew_ref, Wf_ref, bf_ref, filt_ref):
    ew = ew_ref[...]
    d2 = jnp.sum(ew * ew, axis=1, keepdims=True)
    d = jnp.sqrt(d2)
    inv_d = 1.0 / d
    n = jax.lax.broadcasted_iota(jnp.int32, (1, N_RADIAL), 1).astype(jnp.float32)
    n = n + 1.0
    basis = jnp.sin(n * (math.pi / CUTOFF) * d) * inv_d
    cutoff_v = 0.5 * (jnp.cos(math.pi / CUTOFF * d) + 1.0)
    cutoff_v = jnp.where(d < CUTOFF, cutoff_v, 0.0)
    filt = jnp.dot(basis, Wf_ref[...], preferred_element_type=jnp.float32)
    filt_ref[:, 0:D] = (filt + bf_ref[...]) * cutoff_v
    filt_ref[:, D:D + 3] = ew * inv_d
    filt_ref[:, D + 3:FD] = jnp.zeros((ew.shape[0], FD - D - 3), jnp.float32)


def _sc_body(
    filt_hbm, xm_hbm, qpad_hbm, mupad_hbm, ii_hbm, jj_hbm,
    qout_hbm, muout_hbm,
    q_acc, mu_acc, ii0, ii1, jj0, jj1, le, li, lj, e16a, e16b, j16a, j16b,
    f0, f1, xm0, xm1, gsem, csem,
):
    ii_bufs = (ii0, ii1)
    jj_bufs = (jj0, jj1)
    e16s = (e16a, e16b)
    j16s = (j16a, j16b)
    f_bufs = (f0, f1)
    xm_bufs = (xm0, xm1)
    c = lax.axis_index("c")
    s = lax.axis_index("s")
    w = c * NS + s
    iota = lax.broadcasted_iota(jnp.int32, (L,), 0)

    def fire_chunk(ci, slot):
        base = pl.multiple_of(ci * CHUNK, CHUNK)
        pltpu.async_copy(ii_hbm.at[pl.ds(base, CHUNK)], ii_bufs[slot], csem.at[slot])
        pltpu.async_copy(jj_hbm.at[pl.ds(base, CHUNK)], jj_bufs[slot], csem.at[slot])

    def wait_chunk(ci, slot):
        base = pl.multiple_of(ci * CHUNK, CHUNK)
        pltpu.make_async_copy(
            ii_hbm.at[pl.ds(base, CHUNK)], ii_bufs[slot], csem.at[slot]).wait()
        pltpu.make_async_copy(
            jj_hbm.at[pl.ds(base, CHUNK)], jj_bufs[slot], csem.at[slot]).wait()

    def stage_slot(b, slot):
        off = pl.multiple_of(b & CMASK, L)
        e16s[slot][...] = le[pl.ds(off, L)]
        j16s[slot][...] = lj[pl.ds(off, L)]
        pltpu.async_copy(filt_hbm.at[e16s[slot]], f_bufs[slot], gsem.at[slot])
        pltpu.async_copy(xm_hbm.at[j16s[slot]], xm_bufs[slot], gsem.at[slot])

    def stage_any(b):
        par = (b >> 4) & 1

        @pl.when(par == 0)
        def _():
            stage_slot(b, 0)

        @pl.when(par == 1)
        def _():
            stage_slot(b, 1)

    def process_slot(b, slot):
        fb = f_bufs[slot]
        xmb = xm_bufs[slot]
        pltpu.make_async_copy(
            filt_hbm.at[e16s[slot]], fb, gsem.at[slot]).wait()
        pltpu.make_async_copy(
            xm_hbm.at[j16s[slot]], xmb, gsem.at[slot]).wait()
        off = pl.multiple_of(b & CMASK, L)
        ilv = li[pl.ds(off, L)]
        for lane in range(L):
            rowv = jnp.full((L,), ilv[lane])
            dv = fb[lane, pl.ds(D, L)]
            d0 = jnp.full((L,), dv[0])
            d1 = jnp.full((L,), dv[1])
            d2 = jnp.full((L,), dv[2])

            @pl.loop(0, NATOM // L)
            def _feat(k, lane=lane, rowv=rowv,
                      d0=d0, d1=d1, d2=d2, fb=fb, xmb=xmb):
                kL = k * L
                c0 = pl.ds(kL, L)
                c1 = pl.ds(NATOM + kL, L)
                c2 = pl.ds(2 * NATOM + kL, L)
                m0 = pl.ds(D + kL, L)
                m1 = pl.ds(D + NATOM + kL, L)
                m2 = pl.ds(D + 2 * NATOM + kL, L)
                colv = kL + iota
                dq = fb[lane, c0] * xmb[lane, c0]
                plsc.addupdate_scatter(q_acc, [rowv, colv], dq)
                a = fb[lane, c1] * xmb[lane, c1]
                bb = fb[lane, c2] * xmb[lane, c2]
                plsc.addupdate_scatter(
                    mu_acc, [rowv, colv],
                    a * d0 + bb * xmb[lane, m0])
                plsc.addupdate_scatter(
                    mu_acc, [rowv, colv + NATOM],
                    a * d1 + bb * xmb[lane, m1])
                plsc.addupdate_scatter(
                    mu_acc, [rowv, colv + 2 * NATOM],
                    a * d2 + bb * xmb[lane, m2])

    def process_any(b):
        par = (b >> 4) & 1

        @pl.when(par == 0)
        def _():
            process_slot(b, 0)

        @pl.when(par == 1)
        def _():
            process_slot(b, 1)

    def pass_body(p, _0):
        lo = (p * NW + w) * TR
        grow = pl.multiple_of(lo, TR)

        fire_chunk(0, 0)
        pltpu.sync_copy(qpad_hbm.at[pl.ds(grow, TR)], q_acc.at[pl.ds(0, TR)])
        pltpu.sync_copy(mupad_hbm.at[pl.ds(grow, TR)], mu_acc.at[pl.ds(0, TR)])

        def chunk_body(ci, state):
            cntv0, staged0, done0 = state

            def scan_with(slot, cntv0=cntv0, ci=ci):
                wait_chunk(ci, slot)

                @pl.when(ci + 1 < NCHUNK)
                def _():
                    fire_chunk(ci + 1, 1 - slot)

                base = pl.multiple_of(ci * CHUNK, CHUNK)

                def scan_body(k, cntv):
                    o0 = pl.multiple_of(k * (SCU * L), SCU * L)
                    cum = cntv
                    for u in range(SCU):
                        o = o0 + u * L
                        vi = ii_bufs[slot][pl.ds(o, L)]
                        vj = jj_bufs[slot][pl.ds(o, L)]
                        il = vi - lo
                        mask = (il >= 0) & (il < TR)
                        mi = mask.astype(jnp.int32)
                        pos = (cum + plsc.cumsum(mi) - 1) & CMASK
                        ve = base + o + iota
                        plsc.store_scatter(le, [pos], ve, mask=mask)
                        plsc.store_scatter(li, [pos], il, mask=mask)
                        plsc.store_scatter(lj, [pos], vj, mask=mask)
                        cum = cum + plsc.all_reduce_population_count(mask)
                    return cum

                return lax.fori_loop(0, NVEC // SCU, scan_body, cntv0)

            def do_scan(cntv0):
                return lax.cond((ci & 1) == 0,
                                lambda t: scan_with(0, t),
                                lambda t: scan_with(1, t),
                                cntv0)

            def do_pad(cntv0):
                posd = (cntv0 + iota) & CMASK
                plsc.store_scatter(le, [posd], jnp.zeros((L,), jnp.int32))
                plsc.store_scatter(li, [posd], TR + iota)
                plsc.store_scatter(lj, [posd], jnp.zeros((L,), jnp.int32))
                return (cntv0 + L - 1) & ~(L - 1)

            cntv = lax.cond(ci < NCHUNK, do_scan, do_pad, cntv0)
            tot = cntv[0]
            navail = jnp.where(ci < NCHUNK, tot & ~(L - 1), tot)
            nproc = (navail - done0) >> 4

            def drain_iter(t, staged):
                b = done0 + t * L
                prime = staged == b

                @pl.when(prime)
                def _():
                    stage_any(b)

                staged = jnp.where(prime, staged + L, staged)
                pf = (staged == b + L) & (staged < navail)

                @pl.when(pf)
                def _():
                    stage_any(b + L)

                staged = jnp.where(pf, staged + L, staged)
                process_any(b)
                return staged

            staged = lax.fori_loop(0, nproc, drain_iter, staged0)
            return (cntv, staged, navail)

        lax.fori_loop(0, NCHUNK + 1, chunk_body,
                      (jnp.zeros((L,), jnp.int32), jnp.int32(0), jnp.int32(0)))

        pltpu.sync_copy(q_acc.at[pl.ds(0, TR)], qout_hbm.at[pl.ds(grow, TR)])
        pltpu.sync_copy(mu_acc.at[pl.ds(0, TR)], muout_hbm.at[pl.ds(grow, TR)])
        return _0

    lax.fori_loop(0, NPASS, pass_body, jnp.int32(0))


def kernel(q, mu, edge_index, edge_weight, W1, b1, W2, b2, Wf, bf):
    N = q.shape[0]
    E = edge_weight.shape[0]
    BN = 2000
    BE = 4000

    x = pl.pallas_call(
        _mlp_body,
        grid=(N // BN,),
        in_specs=[
            pl.BlockSpec((BN, NATOM), lambda i: (i, 0)),
            pl.BlockSpec((NATOM, NATOM), lambda i: (0, 0)),
            pl.BlockSpec((NATOM,), lambda i: (0,)),
            pl.BlockSpec((NATOM, D), lambda i: (0, 0)),
            pl.BlockSpec((D,), lambda i: (0,)),
        ],
        out_specs=pl.BlockSpec((BN, D), lambda i: (i, 0)),
        out_shape=jax.ShapeDtypeStruct((N, D), jnp.float32),
    )(q, W1, b1, W2, b2)

    filt = pl.pallas_call(
        _filter_body,
        grid=(E // BE,),
        in_specs=[
            pl.BlockSpec((BE, 3), lambda i: (i, 0)),
            pl.BlockSpec((N_RADIAL, D), lambda i: (0, 0)),
            pl.BlockSpec((D,), lambda i: (0,)),
        ],
        out_specs=pl.BlockSpec((BE, FD), lambda i: (i, 0)),
        out_shape=jax.ShapeDtypeStruct((E, FD), jnp.float32),
    )(edge_weight, Wf, bf)

    mu_flat = mu.reshape(N, D)
    xm = jnp.concatenate([x, mu_flat], axis=1)
    qpad = jnp.pad(q, ((0, NPAD - N), (0, 0)))
    mupad = jnp.pad(mu_flat, ((0, NPAD - N), (0, 0)))
    idx_i = edge_index[0]
    idx_j = edge_index[1]

    mesh = plsc.VectorSubcoreMesh(core_axis_name="c", subcore_axis_name="s")
    sc = pl.kernel(
        _sc_body,
        out_type=[
            jax.ShapeDtypeStruct((NPAD, NATOM), jnp.float32),
            jax.ShapeDtypeStruct((NPAD, D), jnp.float32),
        ],
        mesh=mesh,
        compiler_params=pltpu.CompilerParams(needs_layout_passes=False),
        scratch_types=[
            pltpu.VMEM((ACC_ROWS, NATOM), jnp.float32),
            pltpu.VMEM((ACC_ROWS, D), jnp.float32),
            pltpu.VMEM((CHUNK,), jnp.int32),
            pltpu.VMEM((CHUNK,), jnp.int32),
            pltpu.VMEM((CHUNK,), jnp.int32),
            pltpu.VMEM((CHUNK,), jnp.int32),
            pltpu.VMEM((CAP,), jnp.int32),
            pltpu.VMEM((CAP,), jnp.int32),
            pltpu.VMEM((CAP,), jnp.int32),
            pltpu.VMEM((L,), jnp.int32),
            pltpu.VMEM((L,), jnp.int32),
            pltpu.VMEM((L,), jnp.int32),
            pltpu.VMEM((L,), jnp.int32),
            pltpu.VMEM((L, FD), jnp.float32),
            pltpu.VMEM((L, FD), jnp.float32),
            pltpu.VMEM((L, XMD), jnp.float32),
            pltpu.VMEM((L, XMD), jnp.float32),
            pltpu.SemaphoreType.DMA((2,)),
            pltpu.SemaphoreType.DMA((2,)),
        ],
    )
    q_out, mu_out = sc(filt, xm, qpad, mupad, idx_i, idx_j)
    return (q_out[:N], mu_out[:N].reshape(N, 3, NATOM))


# speedup vs baseline: 1.0146x; 1.0146x over previous
import functools
import math

import jax
import jax.numpy as jnp
from jax import lax
from jax.experimental import pallas as pl
from jax.experimental.pallas import tpu as pltpu
from jax.experimental.pallas import tpu_sc as plsc

NATOM = 128
N_RADIAL = 20
CUTOFF = 5.0

N_NODES = 10000
N_EDGES = 320000

NC = 2
NS = 16
L = 16

D = 3 * NATOM
FD = 512
XMD = 2 * D

NW = NC * NS
NPASS = 3
TR = 112
NPAD = NPASS * NW * TR
ACC_ROWS = TR + L
CHUNK = 2000
NVEC = CHUNK // L
NCHUNK = N_EDGES // CHUNK
CAP = 2048
CMASK = CAP - 1
SCU = 5


def _mlp_body(q_ref, W1_ref, b1_ref, W2_ref, b2_ref, x_ref):
    h = jnp.dot(q_ref[...], W1_ref[...], preferred_element_type=jnp.float32)
    h = h + b1_ref[...]
    h = h * jax.nn.sigmoid(h)
    x_ref[...] = (
        jnp.dot(h, W2_ref[...], preferred_element_type=jnp.float32) + b2_ref[...]
    )


def _filter_body(ew_ref, Wf_ref, bf_ref, filt_ref):
    ew = ew_ref[...]
    d2 = jnp.sum(ew * ew, axis=1, keepdims=True)
    d = jnp.sqrt(d2)
    inv_d = 1.0 / d
    n = jax.lax.broadcasted_iota(jnp.int32, (1, N_RADIAL), 1).astype(jnp.float32)
    n = n + 1.0
    basis = jnp.sin(n * (math.pi / CUTOFF) * d) * inv_d
    cutoff_v = 0.5 * (jnp.cos(math.pi / CUTOFF * d) + 1.0)
    cutoff_v = jnp.where(d < CUTOFF, cutoff_v, 0.0)
    filt = jnp.dot(basis, Wf_ref[...], preferred_element_type=jnp.float32)
    filt_ref[:, 0:D] = (filt + bf_ref[...]) * cutoff_v
    filt_ref[:, D:D + 3] = ew * inv_d
    filt_ref[:, D + 3:FD] = jnp.zeros((ew.shape[0], FD - D - 3), jnp.float32)


def _sc_body(
    filt_hbm, xm_hbm, qpad_hbm, mupad_hbm, ii_hbm, jj_hbm,
    qout_hbm, muout_hbm,
    q_acc, mu_acc, ii0, ii1, jj0, jj1, le, li, e16a, e16b, j16a, j16b,
    f0, f1, xm0, xm1, gsem, csem,
):
    ii_bufs = (ii0, ii1)
    jj_bufs = (jj0, jj1)
    e16s = (e16a, e16b)
    j16s = (j16a, j16b)
    f_bufs = (f0, f1)
    xm_bufs = (xm0, xm1)
    c = lax.axis_index("c")
    s = lax.axis_index("s")
    w = c * NS + s
    iota = lax.broadcasted_iota(jnp.int32, (L,), 0)

    def fire_chunk(ci, slot):
        base = pl.multiple_of(ci * CHUNK, CHUNK)
        pltpu.async_copy(ii_hbm.at[pl.ds(base, CHUNK)], ii_bufs[slot], csem.at[slot])
        pltpu.async_copy(jj_hbm.at[pl.ds(base, CHUNK)], jj_bufs[slot], csem.at[slot])

    def wait_chunk(ci, slot):
        base = pl.multiple_of(ci * CHUNK, CHUNK)
        pltpu.make_async_copy(
            ii_hbm.at[pl.ds(base, CHUNK)], ii_bufs[slot], csem.at[slot]).wait()
        pltpu.make_async_copy(
            jj_hbm.at[pl.ds(base, CHUNK)], jj_bufs[slot], csem.at[slot]).wait()

    def stage_slot(b, slot):
        off = pl.multiple_of(b & CMASK, L)
        e16s[slot][...] = le[pl.ds(off, L)]
        j16s[slot][...] = li[pl.ds(off, L)] & ((1 << 14) - 1)
        pltpu.async_copy(filt_hbm.at[e16s[slot]], f_bufs[slot], gsem.at[slot])
        pltpu.async_copy(xm_hbm.at[j16s[slot]], xm_bufs[slot], gsem.at[slot])

    def stage_any(b):
        par = (b >> 4) & 1

        @pl.when(par == 0)
        def _():
            stage_slot(b, 0)

        @pl.when(par == 1)
        def _():
            stage_slot(b, 1)

    def process_slot(b, slot):
        fb = f_bufs[slot]
        xmb = xm_bufs[slot]
        pltpu.make_async_copy(
            filt_hbm.at[e16s[slot]], fb, gsem.at[slot]).wait()
        pltpu.make_async_copy(
            xm_hbm.at[j16s[slot]], xmb, gsem.at[slot]).wait()
        off = pl.multiple_of(b & CMASK, L)
        ilv = li[pl.ds(off, L)] >> 14
        for lane in range(L):
            rowv = jnp.full((L,), ilv[lane])
            dv = fb[lane, pl.ds(D, L)]
            d0 = jnp.full((L,), dv[0])
            d1 = jnp.full((L,), dv[1])
            d2 = jnp.full((L,), dv[2])

            @pl.loop(0, NATOM // L)
            def _feat(k, lane=lane, rowv=rowv,
                      d0=d0, d1=d1, d2=d2, fb=fb, xmb=xmb):
                kL = k * L
                c0 = pl.ds(kL, L)
                c1 = pl.ds(NATOM + kL, L)
                c2 = pl.ds(2 * NATOM + kL, L)
                m0 = pl.ds(D + kL, L)
                m1 = pl.ds(D + NATOM + kL, L)
                m2 = pl.ds(D + 2 * NATOM + kL, L)
                colv = kL + iota
                dq = fb[lane, c0] * xmb[lane, c0]
                plsc.addupdate_scatter(q_acc, [rowv, colv], dq)
                a = fb[lane, c1] * xmb[lane, c1]
                bb = fb[lane, c2] * xmb[lane, c2]
                plsc.addupdate_scatter(
                    mu_acc, [rowv, colv],
                    a * d0 + bb * xmb[lane, m0])
                plsc.addupdate_scatter(
                    mu_acc, [rowv, colv + NATOM],
                    a * d1 + bb * xmb[lane, m1])
                plsc.addupdate_scatter(
                    mu_acc, [rowv, colv + 2 * NATOM],
                    a * d2 + bb * xmb[lane, m2])

    def process_any(b):
        par = (b >> 4) & 1

        @pl.when(par == 0)
        def _():
            process_slot(b, 0)

        @pl.when(par == 1)
        def _():
            process_slot(b, 1)

    def pass_body(p, _0):
        lo = (p * NW + w) * TR
        grow = pl.multiple_of(lo, TR)

        fire_chunk(0, 0)
        pltpu.sync_copy(qpad_hbm.at[pl.ds(grow, TR)], q_acc.at[pl.ds(0, TR)])
        pltpu.sync_copy(mupad_hbm.at[pl.ds(grow, TR)], mu_acc.at[pl.ds(0, TR)])

        def chunk_body(ci, state):
            cntv0, staged0, done0 = state

            def scan_with(slot, cntv0=cntv0, ci=ci):
                wait_chunk(ci, slot)

                @pl.when(ci + 1 < NCHUNK)
                def _():
                    fire_chunk(ci + 1, 1 - slot)

                base = pl.multiple_of(ci * CHUNK, CHUNK)

                def scan_body(k, cntv):
                    o0 = pl.multiple_of(k * (SCU * L), SCU * L)
                    cum = cntv
                    for u in range(SCU):
                        o = o0 + u * L
                        vi = ii_bufs[slot][pl.ds(o, L)]
                        vj = jj_bufs[slot][pl.ds(o, L)]
                        il = vi - lo
                        mask = (il >= 0) & (il < TR)
                        mi = mask.astype(jnp.int32)
                        pos = (cum + plsc.cumsum(mi) - 1) & CMASK
                        ve = base + o + iota
                        plsc.store_scatter(le, [pos], ve, mask=mask)
                        plsc.store_scatter(
                            li, [pos], (il << 14) | vj, mask=mask)
                        cum = cum + plsc.all_reduce_population_count(mask)
                    return cum

                return lax.fori_loop(0, NVEC // SCU, scan_body, cntv0)

            def do_scan(cntv0):
                return lax.cond((ci & 1) == 0,
                                lambda t: scan_with(0, t),
                                lambda t: scan_with(1, t),
                                cntv0)

            def do_pad(cntv0):
                posd = (cntv0 + iota) & CMASK
                plsc.store_scatter(le, [posd], jnp.zeros((L,), jnp.int32))
                plsc.store_scatter(li, [posd], (TR + iota) << 14)
                return (cntv0 + L - 1) & ~(L - 1)

            cntv = lax.cond(ci < NCHUNK, do_scan, do_pad, cntv0)
            tot = cntv[0]
            navail = jnp.where(ci < NCHUNK, tot & ~(L - 1), tot)
            nproc = (navail - done0) >> 4

            def drain_iter(t, staged):
                b = done0 + t * L
                prime = staged == b

                @pl.when(prime)
                def _():
                    stage_any(b)

                staged = jnp.where(prime, staged + L, staged)
                pf = (staged == b + L) & (staged < navail)

                @pl.when(pf)
                def _():
                    stage_any(b + L)

                staged = jnp.where(pf, staged + L, staged)
                process_any(b)
                return staged

            staged = lax.fori_loop(0, nproc, drain_iter, staged0)
            return (cntv, staged, navail)

        lax.fori_loop(0, NCHUNK + 1, chunk_body,
                      (jnp.zeros((L,), jnp.int32), jnp.int32(0), jnp.int32(0)))

        pltpu.sync_copy(q_acc.at[pl.ds(0, TR)], qout_hbm.at[pl.ds(grow, TR)])
        pltpu.sync_copy(mu_acc.at[pl.ds(0, TR)], muout_hbm.at[pl.ds(grow, TR)])
        return _0

    lax.fori_loop(0, NPASS, pass_body, jnp.int32(0))


def kernel(q, mu, edge_index, edge_weight, W1, b1, W2, b2, Wf, bf):
    N = q.shape[0]
    E = edge_weight.shape[0]
    BN = 2000
    BE = 4000

    x = pl.pallas_call(
        _mlp_body,
        grid=(N // BN,),
        in_specs=[
            pl.BlockSpec((BN, NATOM), lambda i: (i, 0)),
            pl.BlockSpec((NATOM, NATOM), lambda i: (0, 0)),
            pl.BlockSpec((NATOM,), lambda i: (0,)),
            pl.BlockSpec((NATOM, D), lambda i: (0, 0)),
            pl.BlockSpec((D,), lambda i: (0,)),
        ],
        out_specs=pl.BlockSpec((BN, D), lambda i: (i, 0)),
        out_shape=jax.ShapeDtypeStruct((N, D), jnp.float32),
    )(q, W1, b1, W2, b2)

    filt = pl.pallas_call(
        _filter_body,
        grid=(E // BE,),
        in_specs=[
            pl.BlockSpec((BE, 3), lambda i: (i, 0)),
            pl.BlockSpec((N_RADIAL, D), lambda i: (0, 0)),
            pl.BlockSpec((D,), lambda i: (0,)),
        ],
        out_specs=pl.BlockSpec((BE, FD), lambda i: (i, 0)),
        out_shape=jax.ShapeDtypeStruct((E, FD), jnp.float32),
    )(edge_weight, Wf, bf)

    mu_flat = mu.reshape(N, D)
    xm = jnp.concatenate([x, mu_flat], axis=1)
    qpad = jnp.pad(q, ((0, NPAD - N), (0, 0)))
    mupad = jnp.pad(mu_flat, ((0, NPAD - N), (0, 0)))
    idx_i = edge_index[0]
    idx_j = edge_index[1]

    mesh = plsc.VectorSubcoreMesh(core_axis_name="c", subcore_axis_name="s")
    sc = pl.kernel(
        _sc_body,
        out_type=[
            jax.ShapeDtypeStruct((NPAD, NATOM), jnp.float32),
            jax.ShapeDtypeStruct((NPAD, D), jnp.float32),
        ],
        mesh=mesh,
        compiler_params=pltpu.CompilerParams(needs_layout_passes=False),
        scratch_types=[
            pltpu.VMEM((ACC_ROWS, NATOM), jnp.float32),
            pltpu.VMEM((ACC_ROWS, D), jnp.float32),
            pltpu.VMEM((CHUNK,), jnp.int32),
            pltpu.VMEM((CHUNK,), jnp.int32),
            pltpu.VMEM((CHUNK,), jnp.int32),
            pltpu.VMEM((CHUNK,), jnp.int32),
            pltpu.VMEM((CAP,), jnp.int32),
            pltpu.VMEM((CAP,), jnp.int32),
            pltpu.VMEM((L,), jnp.int32),
            pltpu.VMEM((L,), jnp.int32),
            pltpu.VMEM((L,), jnp.int32),
            pltpu.VMEM((L,), jnp.int32),
            pltpu.VMEM((L, FD), jnp.float32),
            pltpu.VMEM((L, FD), jnp.float32),
            pltpu.VMEM((L, XMD), jnp.float32),
            pltpu.VMEM((L, XMD), jnp.float32),
            pltpu.SemaphoreType.DMA((2,)),
            pltpu.SemaphoreType.DMA((2,)),
        ],
    )
    q_out, mu_out = sc(filt, xm, qpad, mupad, idx_i, idx_j)
    return (q_out[:N], mu_out[:N].reshape(N, 3, NATOM))
